# baseline (speedup 1.0000x reference)
import functools

import jax
import jax.numpy as jnp
from jax import lax
from jax.experimental import pallas as pl
from jax.experimental.pallas import tpu as pltpu
from jax.experimental.pallas import tpu_sc as plsc

_N = 2048
_D = 256
_K = 64

_NUM_WORKERS = 32
_ROWS_PER_WORKER = _N // _NUM_WORKERS
_BATCH = 8
_LANES = 16

_MASKVAL = -1e30


def _sc_build_s_body(cs_hbm, s_hbm, cs_v, buf, sem):
    wid = lax.axis_index("s") * 2 + lax.axis_index("c")
    row_base = wid * _ROWS_PER_WORKER

    pltpu.sync_copy(cs_hbm.at[pl.ds(row_base * _K, _ROWS_PER_WORKER * _K)],
                    cs_v)

    zeros16 = jnp.zeros((_LANES,), jnp.float32)
    ones16 = jnp.ones((_LANES,), jnp.float32)

    def _zero_chunk(i, _):
        for c in range(8):
            buf[pl.ds((i * 8 + c) * _LANES, _LANES)] = zeros16
        return 0

    lax.fori_loop(0, _BATCH * _N // (8 * _LANES), _zero_chunk, 0)

    def _scatter_batch(b, values):
        for r8 in range(_BATCH):
            for c in range(_K // _LANES):
                cols = cs_v[pl.ds((b * _BATCH + r8) * _K + c * _LANES,
                                  _LANES)]
                plsc.store_scatter(buf, [cols + r8 * _N], values)

    for b in range(_ROWS_PER_WORKER // _BATCH):
        _scatter_batch(b, ones16)
        copy = pltpu.make_async_copy(
            buf,
            s_hbm.at[pl.ds((row_base + b * _BATCH) * _N, _BATCH * _N)],
            sem)
        copy.start()
        copy.wait()
        _scatter_batch(b, zeros16)


@jax.jit
def _sc_build_s(current_samples):
    mesh = plsc.VectorSubcoreMesh(core_axis_name="c", subcore_axis_name="s")
    s_flat = pl.kernel(
        _sc_build_s_body,
        out_type=jax.ShapeDtypeStruct((_N * _N,), jnp.float32),
        mesh=mesh,
        scratch_types=[
            pltpu.VMEM((_ROWS_PER_WORKER * _K,), jnp.int32),
            pltpu.VMEM((_BATCH * _N,), jnp.float32),
            pltpu.SemaphoreType.DMA,
        ],
        compiler_params=pltpu.CompilerParams(needs_layout_passes=False),
    )(current_samples.reshape(-1))
    return s_flat.reshape(_N, _N)


_BR = 256


def _tc_body(x_blk, xt_full, a_blk, s_full, s_own, cs_blk, out_ref):
    f32 = jnp.float32
    sims = lax.dot_general(
        x_blk[...], xt_full[...], (((1,), (0,)), ((), ())),
        preferred_element_type=f32)
    counts = lax.dot_general(
        a_blk[...], s_full[...], (((1,), (0,)), ((), ())),
        preferred_element_type=f32)
    cand = (counts > 0.0) | (s_own[...] > jnp.bfloat16(0.0))
    vals0 = jnp.where(cand, sims, f32(_MASKVAL))

    has_nbrs = jnp.max(a_blk[...], axis=1, keepdims=True) > jnp.bfloat16(0.0)

    iota_col = lax.broadcasted_iota(jnp.int32, (_BR, _N), 1)
    iota_k = lax.broadcasted_iota(jnp.int32, (_BR, _K), 1)

    def body(t, carry):
        vals, out = carry
        idx = lax.argmax(vals, 1, jnp.int32).reshape(_BR, 1)
        vals = jnp.where(iota_col == idx, -jnp.inf, vals)
        out = jnp.where(iota_k == t, idx, out)
        return vals, out

    out0 = jnp.zeros((_BR, _K), jnp.int32)
    _, top = lax.fori_loop(0, _K, body, (vals0, out0))

    out_ref[...] = jnp.where(has_nbrs, top, cs_blk[...])


@jax.jit
def _tc_topk(x, xt, a16, s16, current_samples):
    grid = (_N // _BR,)
    return pl.pallas_call(
        _tc_body,
        grid=grid,
        in_specs=[
            pl.BlockSpec((_BR, _D), lambda i: (i, 0)),
            pl.BlockSpec((_D, _N), lambda i: (0, 0)),
            pl.BlockSpec((_BR, _N), lambda i: (i, 0)),
            pl.BlockSpec((_N, _N), lambda i: (0, 0)),
            pl.BlockSpec((_BR, _N), lambda i: (i, 0)),
            pl.BlockSpec((_BR, _K), lambda i: (i, 0)),
        ],
        out_specs=pl.BlockSpec((_BR, _K), lambda i: (i, 0)),
        out_shape=jax.ShapeDtypeStruct((_N, _K), jnp.int32),
    )(x, xt, a16, s16, s16, current_samples)


def kernel(x, adj, current_samples):
    s = _sc_build_s(current_samples)
    s16 = s.astype(jnp.bfloat16)
    a16 = adj.astype(jnp.bfloat16)
    xt = x.T
    return _tc_topk(x, xt, a16, s16, current_samples)


# speedup vs baseline: 1.0082x; 1.0082x over previous
import functools

import jax
import jax.numpy as jnp
from jax import lax
from jax.experimental import pallas as pl
from jax.experimental.pallas import tpu as pltpu
from jax.experimental.pallas import tpu_sc as plsc

_N = 2048
_D = 256
_K = 64

_NUM_WORKERS = 32
_ROWS_PER_WORKER = _N // _NUM_WORKERS
_BATCH = 8
_LANES = 16

_MASKVAL = -1e30


def _sc_build_s_body(cs_hbm, s_hbm, cs_v, buf, sem):
    wid = lax.axis_index("s") * 2 + lax.axis_index("c")
    row_base = wid * _ROWS_PER_WORKER

    pltpu.sync_copy(cs_hbm.at[pl.ds(row_base * _K, _ROWS_PER_WORKER * _K)],
                    cs_v)

    zeros16 = jnp.zeros((_LANES,), jnp.float32)
    ones16 = jnp.ones((_LANES,), jnp.float32)

    def _zero_chunk(i, _):
        for c in range(8):
            buf[pl.ds((i * 8 + c) * _LANES, _LANES)] = zeros16
        return 0

    lax.fori_loop(0, _BATCH * _N // (8 * _LANES), _zero_chunk, 0)

    def _scatter_batch(b, values):
        for r8 in range(_BATCH):
            for c in range(_K // _LANES):
                cols = cs_v[pl.ds((b * _BATCH + r8) * _K + c * _LANES,
                                  _LANES)]
                plsc.store_scatter(buf, [cols + r8 * _N], values)

    for b in range(_ROWS_PER_WORKER // _BATCH):
        _scatter_batch(b, ones16)
        copy = pltpu.make_async_copy(
            buf,
            s_hbm.at[pl.ds((row_base + b * _BATCH) * _N, _BATCH * _N)],
            sem)
        copy.start()
        copy.wait()
        _scatter_batch(b, zeros16)


@jax.jit
def _sc_build_s(current_samples):
    mesh = plsc.VectorSubcoreMesh(core_axis_name="c", subcore_axis_name="s")
    s_flat = pl.kernel(
        _sc_build_s_body,
        out_type=jax.ShapeDtypeStruct((_N * _N,), jnp.float32),
        mesh=mesh,
        scratch_types=[
            pltpu.VMEM((_ROWS_PER_WORKER * _K,), jnp.int32),
            pltpu.VMEM((_BATCH * _N,), jnp.float32),
            pltpu.SemaphoreType.DMA,
        ],
        compiler_params=pltpu.CompilerParams(needs_layout_passes=False),
    )(current_samples.reshape(-1))
    return s_flat.reshape(_N, _N)


_BR = 256


def _tc_body(x_blk, xt_full, a_blk, s_full, s_own, cs_blk, out_ref):
    f32 = jnp.float32
    sims = lax.dot_general(
        x_blk[...], xt_full[...], (((1,), (0,)), ((), ())),
        preferred_element_type=f32)
    counts = lax.dot_general(
        a_blk[...], s_full[...], (((1,), (0,)), ((), ())),
        preferred_element_type=f32)
    cand = (counts > 0.0) | (s_own[...] > jnp.bfloat16(0.0))
    vals0 = jnp.where(cand, sims, f32(_MASKVAL))

    has_nbrs = jnp.max(a_blk[...], axis=1, keepdims=True) > jnp.bfloat16(0.0)

    iota_col = lax.broadcasted_iota(jnp.int32, (_BR, _N), 1)
    iota_k = lax.broadcasted_iota(jnp.int32, (_BR, _K), 1)

    def body(t, carry):
        vals, m, out = carry
        eq = vals == m
        idx = jnp.min(jnp.where(eq, iota_col, _N), axis=1, keepdims=True)
        vals = jnp.where(iota_col == idx, -jnp.inf, vals)
        m = jnp.max(vals, axis=1, keepdims=True)
        out = jnp.where(iota_k == t, idx, out)
        return vals, m, out

    out0 = jnp.zeros((_BR, _K), jnp.int32)
    m0 = jnp.max(vals0, axis=1, keepdims=True)
    _, _, top = lax.fori_loop(0, _K, body, (vals0, m0, out0))

    out_ref[...] = jnp.where(has_nbrs, top, cs_blk[...])


@jax.jit
def _tc_topk(x, xt, a16, s16, current_samples):
    grid = (_N // _BR,)
    return pl.pallas_call(
        _tc_body,
        grid=grid,
        in_specs=[
            pl.BlockSpec((_BR, _D), lambda i: (i, 0)),
            pl.BlockSpec((_D, _N), lambda i: (0, 0)),
            pl.BlockSpec((_BR, _N), lambda i: (i, 0)),
            pl.BlockSpec((_N, _N), lambda i: (0, 0)),
            pl.BlockSpec((_BR, _N), lambda i: (i, 0)),
            pl.BlockSpec((_BR, _K), lambda i: (i, 0)),
        ],
        out_specs=pl.BlockSpec((_BR, _K), lambda i: (i, 0)),
        out_shape=jax.ShapeDtypeStruct((_N, _K), jnp.int32),
    )(x, xt, a16, s16, s16, current_samples)


def kernel(x, adj, current_samples):
    s = _sc_build_s(current_samples)
    s16 = s.astype(jnp.bfloat16)
    a16 = adj.astype(jnp.bfloat16)
    xt = x.T
    return _tc_topk(x, xt, a16, s16, current_samples)


# speedup vs baseline: 1.2925x; 1.2820x over previous
import functools

import jax
import jax.numpy as jnp
from jax import lax
from jax.experimental import pallas as pl
from jax.experimental.pallas import tpu as pltpu
from jax.experimental.pallas import tpu_sc as plsc

_N = 2048
_D = 256
_K = 64

_NUM_WORKERS = 32
_ROWS_PER_WORKER = _N // _NUM_WORKERS
_BATCH = 8
_LANES = 16

_MASKVAL = -1e30
_MININT = jnp.int32(-2147483648)


def _sc_build_s_body(cs_hbm, s_hbm, cs_v, buf, sem):
    wid = lax.axis_index("s") * 2 + lax.axis_index("c")
    row_base = wid * _ROWS_PER_WORKER

    pltpu.sync_copy(cs_hbm.at[pl.ds(row_base * _K, _ROWS_PER_WORKER * _K)],
                    cs_v)

    zeros16 = jnp.zeros((_LANES,), jnp.float32)
    ones16 = jnp.ones((_LANES,), jnp.float32)

    def _zero_chunk(i, _):
        for c in range(8):
            buf[pl.ds((i * 8 + c) * _LANES, _LANES)] = zeros16
        return 0

    lax.fori_loop(0, _BATCH * _N // (8 * _LANES), _zero_chunk, 0)

    def _scatter_batch(b, values):
        for r8 in range(_BATCH):
            for c in range(_K // _LANES):
                cols = cs_v[pl.ds((b * _BATCH + r8) * _K + c * _LANES,
                                  _LANES)]
                plsc.store_scatter(buf, [cols + r8 * _N], values)

    for b in range(_ROWS_PER_WORKER // _BATCH):
        _scatter_batch(b, ones16)
        copy = pltpu.make_async_copy(
            buf,
            s_hbm.at[pl.ds((row_base + b * _BATCH) * _N, _BATCH * _N)],
            sem)
        copy.start()
        copy.wait()
        _scatter_batch(b, zeros16)


@jax.jit
def _sc_build_s(current_samples):
    mesh = plsc.VectorSubcoreMesh(core_axis_name="c", subcore_axis_name="s")
    s_flat = pl.kernel(
        _sc_build_s_body,
        out_type=jax.ShapeDtypeStruct((_N * _N,), jnp.float32),
        mesh=mesh,
        scratch_types=[
            pltpu.VMEM((_ROWS_PER_WORKER * _K,), jnp.int32),
            pltpu.VMEM((_BATCH * _N,), jnp.float32),
            pltpu.SemaphoreType.DMA,
        ],
        compiler_params=pltpu.CompilerParams(needs_layout_passes=False),
    )(current_samples.reshape(-1))
    return s_flat.reshape(_N, _N)


_BR = 256


def _tc_mask_body(x_blk, xt_full, a_blk, s_full, s_own, masked_ref, hnb_ref):
    f32 = jnp.float32
    sims = lax.dot_general(
        x_blk[...], xt_full[...], (((1,), (0,)), ((), ())),
        preferred_element_type=f32)
    counts = lax.dot_general(
        a_blk[...], s_full[...], (((1,), (0,)), ((), ())),
        preferred_element_type=f32)
    cand = (counts > 0.0) | (s_own[...] > jnp.bfloat16(0.0))
    masked_ref[...] = jnp.where(cand, sims, f32(_MASKVAL))
    hnb = jnp.max(a_blk[...], axis=1, keepdims=True) > jnp.bfloat16(0.0)
    hnb_ref[...] = jnp.broadcast_to(hnb.astype(jnp.int32), (_BR, _LANES))


@jax.jit
def _tc_mask(x, xt, a16, s16):
    grid = (_N // _BR,)
    return pl.pallas_call(
        _tc_mask_body,
        grid=grid,
        in_specs=[
            pl.BlockSpec((_BR, _D), lambda i: (i, 0)),
            pl.BlockSpec((_D, _N), lambda i: (0, 0)),
            pl.BlockSpec((_BR, _N), lambda i: (i, 0)),
            pl.BlockSpec((_N, _N), lambda i: (0, 0)),
            pl.BlockSpec((_BR, _N), lambda i: (i, 0)),
        ],
        out_specs=[
            pl.BlockSpec((_BR, _N), lambda i: (i, 0)),
            pl.BlockSpec((_BR, _LANES), lambda i: (i, 0)),
        ],
        out_shape=[
            jax.ShapeDtypeStruct((_N, _N), jnp.float32),
            jax.ShapeDtypeStruct((_N, _LANES), jnp.int32),
        ],
    )(x, xt, a16, s16, s16)


def _sortable(v):
    b = lax.bitcast_convert_type(v, jnp.int32)
    return b ^ (lax.shift_right_arithmetic(b, 31) | _MININT)


def _perm16(v, idx):
    return lax.gather(
        v, idx[:, None],
        lax.GatherDimensionNumbers(offset_dims=(), collapsed_slice_dims=(0,),
                                   start_index_map=(0,)),
        (1,), mode=lax.GatherScatterMode.PROMISE_IN_BOUNDS)


def _sc_topk_body(vals_hbm, hnb_hbm, cs_hbm, out_hbm,
                  rowbuf, eva, eia, evb, eib, o_v, o_i, hist,
                  cs_v, hnb_v, outb, sem):
    iota = lax.broadcasted_iota(jnp.int32, (_LANES,), 0)
    one16 = jnp.ones((_LANES,), jnp.int32)
    zero16 = jnp.zeros((_LANES,), jnp.int32)

    wid = lax.axis_index("s") * 2 + lax.axis_index("c")
    base = wid * _ROWS_PER_WORKER

    pltpu.sync_copy(cs_hbm.at[pl.ds(base * _K, _ROWS_PER_WORKER * _K)], cs_v)
    pltpu.sync_copy(
        hnb_hbm.at[pl.ds(base * _LANES, _ROWS_PER_WORKER * _LANES)], hnb_v)

    pltpu.make_async_copy(vals_hbm.at[pl.ds(base * _N, _N)],
                          rowbuf.at[pl.ds(0, _N)], sem).start()
    pltpu.make_async_copy(vals_hbm.at[pl.ds((base + 1) * _N, _N)],
                          rowbuf.at[pl.ds(_N, _N)], sem).start()

    def _scan16(hslice, need):
        csum = plsc.cumsum(lax.rev(hslice, (0,)))
        cand = jnp.where(csum >= need, 15 - iota, -1)
        bstar = jnp.max(cand)
        hb = jnp.sum(jnp.where(iota == bstar, hslice, 0))
        sab = jnp.sum(jnp.where(iota >= bstar, hslice, 0))
        exact = sab == need
        return bstar, exact

    def row_body(r, _):
        off = (r & 1) * _N

        pltpu.make_async_copy(vals_hbm.at[pl.ds(0, _N)],
                              rowbuf.at[pl.ds(off, _N)], sem).wait()

        for c in range(16):
            hist[pl.ds(c * 16, 16)] = zero16

        def h1(c, carry):
            for u in range(4):
                v = rowbuf[pl.ds(off + (c * 4 + u) * 16, 16)]
                f = lax.shift_right_logical(_sortable(v), 24)
                plsc.addupdate_scatter(hist, [f], one16)
            return carry

        lax.fori_loop(0, 32, h1, 0)

        need = jnp.int32(_K)
        carry = jnp.int32(0)
        bstar = jnp.int32(-1)
        for i in reversed(range(16)):
            h = hist[pl.ds(i * 16, 16)]
            csum = plsc.cumsum(lax.rev(h, (0,))) + carry
            cand = jnp.where(csum >= need, (i * 16 + 15) - iota, -1)
            bstar = jnp.maximum(bstar, jnp.max(cand))
            carry = carry + jnp.sum(h)
        hb = jnp.int32(0)
        sab = jnp.int32(0)
        for i in range(16):
            h = hist[pl.ds(i * 16, 16)]
            ids = iota + i * 16
            hb = hb + jnp.sum(jnp.where(ids == bstar, h, 0))
            sab = sab + jnp.sum(jnp.where(ids >= bstar, h, 0))
        exact = sab == need
        bhi = jnp.where(exact, bstar - 1, bstar)

        def ex1(c, cst):
            cd, cb = cst
            v = rowbuf[pl.ds(off + c * 16, 16)]
            oi = iota + c * 16
            f = lax.shift_right_logical(_sortable(v), 24)
            dm = f > bhi
            bm = (f == bstar) & jnp.logical_not(exact)
            dmi = dm.astype(jnp.int32)
            bmi = bm.astype(jnp.int32)
            pd = plsc.cumsum(dmi) - 1 + cd
            pb = plsc.cumsum(bmi) - 1 + cb
            plsc.store_scatter(o_v, [pd], v, mask=dm)
            plsc.store_scatter(o_i, [pd], oi, mask=dm)
            plsc.store_scatter(eva, [pb], v, mask=bm)
            plsc.store_scatter(eia, [pb], oi, mask=bm)
            return cd + jnp.sum(dmi), cb + jnp.sum(bmi)

        nf, nb = lax.fori_loop(0, _N // 16, ex1, (jnp.int32(0), jnp.int32(0)))

        def _level(sv, si, dv, di, sh, nf, nb):
            hist[pl.ds(0, 16)] = zero16

            def hh(c, carry):
                m = (iota + c * 16) < nb
                v = sv[pl.ds(c * 16, 16)]
                f = lax.shift_right_logical(_sortable(v), sh) & 15
                plsc.addupdate_scatter(hist, [f], one16, mask=m)
                return carry

            lax.fori_loop(0, (nb + 15) // 16, hh, 0)
            need_l = jnp.int32(_K) - nf
            bstar, exact = _scan16(hist[pl.ds(0, 16)], need_l)
            bhi = jnp.where(exact, bstar - 1, bstar)

            def ee(c, cst):
                cd, cb = cst
                m = (iota + c * 16) < nb
                v = sv[pl.ds(c * 16, 16)]
                oi = si[pl.ds(c * 16, 16)]
                f = lax.shift_right_logical(_sortable(v), sh) & 15
                dm = (f > bhi) & m
                bm = (f == bstar) & jnp.logical_not(exact) & m
                dmi = dm.astype(jnp.int32)
                bmi = bm.astype(jnp.int32)
                pd = plsc.cumsum(dmi) - 1 + cd
                pb = plsc.cumsum(bmi) - 1 + cb
                plsc.store_scatter(o_v, [pd], v, mask=dm)
                plsc.store_scatter(o_i, [pd], oi, mask=dm)
                plsc.store_scatter(dv, [pb], v, mask=bm)
                plsc.store_scatter(di, [pb], oi, mask=bm)
                return cd + jnp.sum(dmi), cb + jnp.sum(bmi)

            return lax.fori_loop(0, (nb + 15) // 16, ee, (nf, jnp.int32(0)))

        for lvl in range(6):
            sh = 20 - lvl * 4
            if lvl % 2 == 0:
                nf, nb = _level(eva, eia, evb, eib, sh, nf, nb)
            else:
                nf, nb = _level(evb, eib, eva, eia, sh, nf, nb)

        need_f = jnp.int32(_K) - nf

        def ap(c, carry):
            ordv = iota + c * 16
            m = (ordv < need_f) & (ordv < nb)
            v = eva[pl.ds(c * 16, 16)]
            oi = eia[pl.ds(c * 16, 16)]
            plsc.store_scatter(o_v, [nf + ordv], v, mask=m)
            plsc.store_scatter(o_i, [nf + ordv], oi, mask=m)
            return carry

        lax.fori_loop(0, (nb + 15) // 16, ap, 0)

        va = [o_v[pl.ds(q * 16, 16)] for q in range(4)]
        ia = [o_i[pl.ds(q * 16, 16)] for q in range(4)]

        def before(v1, i1, v2, i2):
            return (v1 > v2) | ((v1 == v2) & (i1 < i2))

        for k in (2, 4, 8, 16, 32, 64):
            d = k // 2
            while d >= 1:
                if d >= 16:
                    dv_ = d // 16
                    for vr in range(4):
                        if vr & dv_:
                            continue
                        vr2 = vr | dv_
                        asc = ((vr * 16) & k) != 0
                        bef = before(va[vr], ia[vr], va[vr2], ia[vr2])
                        ts = jnp.logical_xor(bef, asc)
                        nav = jnp.where(ts, va[vr], va[vr2])
                        nai = jnp.where(ts, ia[vr], ia[vr2])
                        nbv = jnp.where(ts, va[vr2], va[vr])
                        nbi = jnp.where(ts, ia[vr2], ia[vr])
                        va[vr], ia[vr] = nav, nai
                        va[vr2], ia[vr2] = nbv, nbi
                else:
                    pidx = iota ^ d
                    is_right = (iota & d) != 0
                    for vr in range(4):
                        if k >= 16:
                            asc_arr = jnp.full((_LANES,),
                                               ((vr * 16) & k) != 0, jnp.bool_)
                        else:
                            asc_arr = (iota & k) != 0
                        pv = _perm16(va[vr], pidx)
                        pi = _perm16(ia[vr], pidx)
                        bef = before(va[vr], ia[vr], pv, pi)
                        ts = jnp.logical_xor(jnp.logical_xor(bef, asc_arr),
                                             is_right)
                        va[vr] = jnp.where(ts, va[vr], pv)
                        ia[vr] = jnp.where(ts, ia[vr], pi)
                d //= 2

        flag16 = hnb_v[pl.ds(r * _LANES, _LANES)]
        for q in range(4):
            csq = cs_v[pl.ds(r * _K + q * 16, 16)]
            outb[pl.ds(r * _K + q * 16, 16)] = jnp.where(flag16 != 0, ia[q],
                                                         csq)

        @pl.when(r < _ROWS_PER_WORKER - 2)
        def _():
            nxt = (base + r + 2) * _N
            pltpu.make_async_copy(vals_hbm.at[pl.ds(nxt, _N)],
                                  rowbuf.at[pl.ds(off, _N)], sem).start()

        return 0

    lax.fori_loop(0, _ROWS_PER_WORKER, row_body, 0)
    pltpu.sync_copy(outb, out_hbm.at[pl.ds(base * _K, _ROWS_PER_WORKER * _K)])


@jax.jit
def _sc_topk(masked, hnb, current_samples):
    mesh = plsc.VectorSubcoreMesh(core_axis_name="c", subcore_axis_name="s")
    out_flat = pl.kernel(
        _sc_topk_body,
        out_type=jax.ShapeDtypeStruct((_N * _K,), jnp.int32),
        mesh=mesh,
        scratch_types=[
            pltpu.VMEM((2 * _N,), jnp.float32),
            pltpu.VMEM((_N,), jnp.float32),
            pltpu.VMEM((_N,), jnp.int32),
            pltpu.VMEM((_N,), jnp.float32),
            pltpu.VMEM((_N,), jnp.int32),
            pltpu.VMEM((_K,), jnp.float32),
            pltpu.VMEM((_K,), jnp.int32),
            pltpu.VMEM((256,), jnp.int32),
            pltpu.VMEM((_ROWS_PER_WORKER * _K,), jnp.int32),
            pltpu.VMEM((_ROWS_PER_WORKER * _LANES,), jnp.int32),
            pltpu.VMEM((_ROWS_PER_WORKER * _K,), jnp.int32),
            pltpu.SemaphoreType.DMA,
        ],
        compiler_params=pltpu.CompilerParams(needs_layout_passes=False),
    )(masked.reshape(-1), hnb.reshape(-1), current_samples.reshape(-1))
    return out_flat.reshape(_N, _K)


def kernel(x, adj, current_samples):
    s = _sc_build_s(current_samples)
    s16 = s.astype(jnp.bfloat16)
    a16 = adj.astype(jnp.bfloat16)
    xt = x.T
    masked, hnb = _tc_mask(x, xt, a16, s16)
    return _sc_topk(masked, hnb, current_samples)


# speedup vs baseline: 1.4784x; 1.1437x over previous
import functools

import jax
import jax.numpy as jnp
from jax import lax
from jax.experimental import pallas as pl
from jax.experimental.pallas import tpu as pltpu
from jax.experimental.pallas import tpu_sc as plsc

_N = 2048
_D = 256
_K = 64

_NUM_WORKERS = 32
_ROWS_PER_WORKER = _N // _NUM_WORKERS
_BATCH = 8
_LANES = 16

_MASKVAL = -1e30
_MININT = jnp.int32(-2147483648)


def _sc_build_s_body(cs_hbm, s_hbm, cs_v, buf, sem):
    wid = lax.axis_index("s") * 2 + lax.axis_index("c")
    row_base = wid * _ROWS_PER_WORKER

    pltpu.sync_copy(cs_hbm.at[pl.ds(row_base * _K, _ROWS_PER_WORKER * _K)],
                    cs_v)

    zeros16 = jnp.zeros((_LANES,), jnp.float32)
    ones16 = jnp.ones((_LANES,), jnp.float32)

    def _zero_chunk(i, _):
        for c in range(8):
            buf[pl.ds((i * 8 + c) * _LANES, _LANES)] = zeros16
        return 0

    lax.fori_loop(0, _BATCH * _N // (8 * _LANES), _zero_chunk, 0)

    def _scatter_batch(b, values):
        for r8 in range(_BATCH):
            for c in range(_K // _LANES):
                cols = cs_v[pl.ds((b * _BATCH + r8) * _K + c * _LANES,
                                  _LANES)]
                plsc.store_scatter(buf, [cols + r8 * _N], values)

    for b in range(_ROWS_PER_WORKER // _BATCH):
        _scatter_batch(b, ones16)
        copy = pltpu.make_async_copy(
            buf,
            s_hbm.at[pl.ds((row_base + b * _BATCH) * _N, _BATCH * _N)],
            sem)
        copy.start()
        copy.wait()
        _scatter_batch(b, zeros16)


@jax.jit
def _sc_build_s(current_samples):
    mesh = plsc.VectorSubcoreMesh(core_axis_name="c", subcore_axis_name="s")
    s_flat = pl.kernel(
        _sc_build_s_body,
        out_type=jax.ShapeDtypeStruct((_N * _N,), jnp.float32),
        mesh=mesh,
        scratch_types=[
            pltpu.VMEM((_ROWS_PER_WORKER * _K,), jnp.int32),
            pltpu.VMEM((_BATCH * _N,), jnp.float32),
            pltpu.SemaphoreType.DMA,
        ],
        compiler_params=pltpu.CompilerParams(needs_layout_passes=False),
    )(current_samples.reshape(-1))
    return s_flat.reshape(_N, _N)


_BR = 256


def _tc_mask_body(x_blk, xt_full, a_blk, s_full, s_own, masked_ref, hnb_ref):
    f32 = jnp.float32
    sims = lax.dot_general(
        x_blk[...], xt_full[...], (((1,), (0,)), ((), ())),
        preferred_element_type=f32)
    counts = lax.dot_general(
        a_blk[...], s_full[...], (((1,), (0,)), ((), ())),
        preferred_element_type=f32)
    cand = (counts > 0.0) | (s_own[...] > jnp.bfloat16(0.0))
    masked_ref[...] = jnp.where(cand, sims, f32(_MASKVAL))
    hnb = jnp.max(a_blk[...], axis=1, keepdims=True) > jnp.bfloat16(0.0)
    hnb_ref[...] = jnp.broadcast_to(hnb.astype(jnp.int32), (_BR, _LANES))


@jax.jit
def _tc_mask(x, xt, a16, s16):
    grid = (_N // _BR,)
    return pl.pallas_call(
        _tc_mask_body,
        grid=grid,
        in_specs=[
            pl.BlockSpec((_BR, _D), lambda i: (i, 0)),
            pl.BlockSpec((_D, _N), lambda i: (0, 0)),
            pl.BlockSpec((_BR, _N), lambda i: (i, 0)),
            pl.BlockSpec((_N, _N), lambda i: (0, 0)),
            pl.BlockSpec((_BR, _N), lambda i: (i, 0)),
        ],
        out_specs=[
            pl.BlockSpec((_BR, _N), lambda i: (i, 0)),
            pl.BlockSpec((_BR, _LANES), lambda i: (i, 0)),
        ],
        out_shape=[
            jax.ShapeDtypeStruct((_N, _N), jnp.float32),
            jax.ShapeDtypeStruct((_N, _LANES), jnp.int32),
        ],
    )(x, xt, a16, s16, s16)


def _sortable(v):
    b = lax.bitcast_convert_type(v, jnp.int32)
    return b ^ (lax.shift_right_arithmetic(b, 31) | _MININT)


def _perm16(v, idx):
    return lax.gather(
        v, idx[:, None],
        lax.GatherDimensionNumbers(offset_dims=(), collapsed_slice_dims=(0,),
                                   start_index_map=(0,)),
        (1,), mode=lax.GatherScatterMode.PROMISE_IN_BOUNDS)


def _sc_topk_body(vals_hbm, hnb_hbm, cs_hbm, out_hbm,
                  rowbuf, eva, eia, evb, eib, o_v, o_i, hist,
                  cs_v, hnb_v, outb, sem):
    iota = lax.broadcasted_iota(jnp.int32, (_LANES,), 0)
    one16 = jnp.ones((_LANES,), jnp.int32)
    zero16 = jnp.zeros((_LANES,), jnp.int32)

    wid = lax.axis_index("s") * 2 + lax.axis_index("c")
    base = wid * _ROWS_PER_WORKER

    pltpu.sync_copy(cs_hbm.at[pl.ds(base * _K, _ROWS_PER_WORKER * _K)], cs_v)
    pltpu.sync_copy(
        hnb_hbm.at[pl.ds(base * _LANES, _ROWS_PER_WORKER * _LANES)], hnb_v)

    pltpu.make_async_copy(vals_hbm.at[pl.ds(base * _N, _N)],
                          rowbuf.at[pl.ds(0, _N)], sem).start()
    pltpu.make_async_copy(vals_hbm.at[pl.ds((base + 1) * _N, _N)],
                          rowbuf.at[pl.ds(_N, _N)], sem).start()

    BIG = jnp.int32(2147483647)
    fifteen = jnp.full((_LANES,), 15, jnp.int32)

    def _scan16(hslice, need_v):
        csum = plsc.cumsum(lax.rev(hslice, (0,)))
        cond = csum >= need_v
        bstar = jnp.max(jnp.where(cond, 15 - iota, -1))
        sab = jnp.min(jnp.where(cond, csum, BIG))
        exact = sab == jnp.max(need_v)
        return bstar, exact

    def row_body(r, _):
        off = (r & 1) * _N

        pltpu.make_async_copy(vals_hbm.at[pl.ds(0, _N)],
                              rowbuf.at[pl.ds(off, _N)], sem).wait()

        for c in range(16):
            hist[pl.ds(c * 16, 16)] = zero16

        def h1(c, carry):
            for u in range(4):
                v = rowbuf[pl.ds(off + (c * 4 + u) * 16, 16)]
                f = lax.shift_right_logical(_sortable(v), 24)
                plsc.addupdate_scatter(hist, [f], one16)
            return carry

        lax.fori_loop(0, 32, h1, 0)

        kneed = jnp.int32(_K)
        carry_v = zero16
        acc_b = jnp.full((_LANES,), -1, jnp.int32)
        acc_sab = jnp.full((_LANES,), 2147483647, jnp.int32)
        acc_s1 = zero16
        for i in reversed(range(16)):
            h = hist[pl.ds(i * 16, 16)]
            csum = plsc.cumsum(lax.rev(h, (0,))) + carry_v
            cond = csum >= kneed
            acc_b = jnp.maximum(acc_b, jnp.where(cond, (i * 16 + 15) - iota,
                                                 -1))
            acc_sab = jnp.minimum(acc_sab, jnp.where(cond, csum, BIG))
            acc_s1 = jnp.maximum(acc_s1, jnp.where(cond, 0, csum))
            carry_v = _perm16(csum, fifteen)
        bstar = jnp.max(acc_b)
        sab = jnp.min(acc_sab)
        s1 = jnp.max(acc_s1)
        exact = sab == kneed
        bhi = jnp.where(exact, bstar - 1, bstar)

        def ex1(c, cst):
            cd, cb = cst
            v = rowbuf[pl.ds(off + c * 16, 16)]
            oi = iota + c * 16
            f = lax.shift_right_logical(_sortable(v), 24)
            dm = f > bhi
            bm = (f == bstar) & jnp.logical_not(exact)
            dmi = dm.astype(jnp.int32)
            bmi = bm.astype(jnp.int32)
            pd = plsc.cumsum(dmi) - 1 + cd
            pb = plsc.cumsum(bmi) - 1 + cb
            plsc.store_scatter(o_v, [pd], v, mask=dm)
            plsc.store_scatter(o_i, [pd], oi, mask=dm)
            plsc.store_scatter(eva, [pb], v, mask=bm)
            plsc.store_scatter(eia, [pb], oi, mask=bm)
            return _perm16(pd, fifteen) + 1, _perm16(pb, fifteen) + 1

        nf_v, nb_v = lax.fori_loop(0, _N // 16, ex1, (zero16, zero16))
        nf = nf_v
        nb = jnp.max(nb_v)

        def _level(sv, si, dv, di, sh, nf, nb):
            hist[pl.ds(0, 16)] = zero16

            def hh(c, carry):
                m = (iota + c * 16) < nb
                v = sv[pl.ds(c * 16, 16)]
                f = lax.shift_right_logical(_sortable(v), sh) & 15
                plsc.addupdate_scatter(hist, [f], one16, mask=m)
                return carry

            lax.fori_loop(0, (nb + 15) // 16, hh, 0)
            need_l = jnp.int32(_K) - nf
            bstar, exact = _scan16(hist[pl.ds(0, 16)], need_l)
            bhi = jnp.where(exact, bstar - 1, bstar)
            del need_l

            def ee(c, cst):
                cd, cb = cst
                m = (iota + c * 16) < nb
                v = sv[pl.ds(c * 16, 16)]
                oi = si[pl.ds(c * 16, 16)]
                f = lax.shift_right_logical(_sortable(v), sh) & 15
                dm = (f > bhi) & m
                bm = (f == bstar) & jnp.logical_not(exact) & m
                dmi = dm.astype(jnp.int32)
                bmi = bm.astype(jnp.int32)
                pd = plsc.cumsum(dmi) - 1 + cd
                pb = plsc.cumsum(bmi) - 1 + cb
                plsc.store_scatter(o_v, [pd], v, mask=dm)
                plsc.store_scatter(o_i, [pd], oi, mask=dm)
                plsc.store_scatter(dv, [pb], v, mask=bm)
                plsc.store_scatter(di, [pb], oi, mask=bm)
                return _perm16(pd, fifteen) + 1, _perm16(pb, fifteen) + 1

            nf2, nb2 = lax.fori_loop(0, (nb + 15) // 16, ee, (nf, zero16))
            return nf2, jnp.max(nb2)

        for lvl in range(6):
            sh = 20 - lvl * 4
            if lvl % 2 == 0:
                nf, nb = _level(eva, eia, evb, eib, sh, nf, nb)
            else:
                nf, nb = _level(evb, eib, eva, eia, sh, nf, nb)

        need_f = jnp.int32(_K) - nf

        def ap(c, carry):
            ordv = iota + c * 16
            m = (ordv < need_f) & (ordv < nb)
            v = eva[pl.ds(c * 16, 16)]
            oi = eia[pl.ds(c * 16, 16)]
            plsc.store_scatter(o_v, [nf + ordv], v, mask=m)
            plsc.store_scatter(o_i, [nf + ordv], oi, mask=m)
            return carry

        lax.fori_loop(0, (nb + 15) // 16, ap, 0)

        va = [o_v[pl.ds(q * 16, 16)] for q in range(4)]
        ia = [o_i[pl.ds(q * 16, 16)] for q in range(4)]

        def before(v1, i1, v2, i2):
            return (v1 > v2) | ((v1 == v2) & (i1 < i2))

        for k in (2, 4, 8, 16, 32, 64):
            d = k // 2
            while d >= 1:
                if d >= 16:
                    dv_ = d // 16
                    for vr in range(4):
                        if vr & dv_:
                            continue
                        vr2 = vr | dv_
                        asc = ((vr * 16) & k) != 0
                        bef = before(va[vr], ia[vr], va[vr2], ia[vr2])
                        ts = jnp.logical_xor(bef, asc)
                        nav = jnp.where(ts, va[vr], va[vr2])
                        nai = jnp.where(ts, ia[vr], ia[vr2])
                        nbv = jnp.where(ts, va[vr2], va[vr])
                        nbi = jnp.where(ts, ia[vr2], ia[vr])
                        va[vr], ia[vr] = nav, nai
                        va[vr2], ia[vr2] = nbv, nbi
                else:
                    pidx = iota ^ d
                    is_right = (iota & d) != 0
                    for vr in range(4):
                        if k >= 16:
                            asc_arr = jnp.full((_LANES,),
                                               ((vr * 16) & k) != 0, jnp.bool_)
                        else:
                            asc_arr = (iota & k) != 0
                        pv = _perm16(va[vr], pidx)
                        pi = _perm16(ia[vr], pidx)
                        bef = before(va[vr], ia[vr], pv, pi)
                        ts = jnp.logical_xor(jnp.logical_xor(bef, asc_arr),
                                             is_right)
                        va[vr] = jnp.where(ts, va[vr], pv)
                        ia[vr] = jnp.where(ts, ia[vr], pi)
                d //= 2

        flag16 = hnb_v[pl.ds(r * _LANES, _LANES)]
        for q in range(4):
            csq = cs_v[pl.ds(r * _K + q * 16, 16)]
            outb[pl.ds(r * _K + q * 16, 16)] = jnp.where(flag16 != 0, ia[q],
                                                         csq)

        @pl.when(r < _ROWS_PER_WORKER - 2)
        def _():
            nxt = (base + r + 2) * _N
            pltpu.make_async_copy(vals_hbm.at[pl.ds(nxt, _N)],
                                  rowbuf.at[pl.ds(off, _N)], sem).start()

        return 0

    lax.fori_loop(0, _ROWS_PER_WORKER, row_body, 0)
    pltpu.sync_copy(outb, out_hbm.at[pl.ds(base * _K, _ROWS_PER_WORKER * _K)])


@jax.jit
def _sc_topk(masked, hnb, current_samples):
    mesh = plsc.VectorSubcoreMesh(core_axis_name="c", subcore_axis_name="s")
    out_flat = pl.kernel(
        _sc_topk_body,
        out_type=jax.ShapeDtypeStruct((_N * _K,), jnp.int32),
        mesh=mesh,
        scratch_types=[
            pltpu.VMEM((2 * _N,), jnp.float32),
            pltpu.VMEM((_N,), jnp.float32),
            pltpu.VMEM((_N,), jnp.int32),
            pltpu.VMEM((_N,), jnp.float32),
            pltpu.VMEM((_N,), jnp.int32),
            pltpu.VMEM((_K,), jnp.float32),
            pltpu.VMEM((_K,), jnp.int32),
            pltpu.VMEM((256,), jnp.int32),
            pltpu.VMEM((_ROWS_PER_WORKER * _K,), jnp.int32),
            pltpu.VMEM((_ROWS_PER_WORKER * _LANES,), jnp.int32),
            pltpu.VMEM((_ROWS_PER_WORKER * _K,), jnp.int32),
            pltpu.SemaphoreType.DMA,
        ],
        compiler_params=pltpu.CompilerParams(needs_layout_passes=False),
    )(masked.reshape(-1), hnb.reshape(-1), current_samples.reshape(-1))
    return out_flat.reshape(_N, _K)


def kernel(x, adj, current_samples):
    s = _sc_build_s(current_samples)
    s16 = s.astype(jnp.bfloat16)
    a16 = adj.astype(jnp.bfloat16)
    xt = x.T
    masked, hnb = _tc_mask(x, xt, a16, s16)
    return _sc_topk(masked, hnb, current_samples)
